# baseline (speedup 1.0000x reference)
import functools

import jax
import jax.numpy as jnp
from jax import lax
from jax.experimental import pallas as pl
from jax.experimental.pallas import tpu as pltpu
from jax.experimental.pallas import tpu_sc as plsc

_B = 16384
_DIM = 16
_VOCAB1 = 1001
_NC = 2
_NS = 16
_NW = _NC * _NS
_BPW = _B // _NW
_INV = 1.0 / (1.0 + 1e-6)
_BLK = 2048


def _sc_body(table_hbm, idx1_hbm, idx2_hbm, emb_hbm, rest_hbm,
             idx1_v, idx2_v, rows1_v, rows2_v, sem1, sem2):
    wid = lax.axis_index("s") * _NC + lax.axis_index("c")
    base = wid * _BPW
    pltpu.sync_copy(idx1_hbm.at[pl.ds(base, _BPW)], idx1_v)
    pltpu.sync_copy(idx2_hbm.at[pl.ds(base, _BPW)], idx2_v)
    c1 = pltpu.async_copy(table_hbm.at[idx1_v], rows1_v, sem1)
    c2 = pltpu.async_copy(table_hbm.at[idx2_v], rows2_v, sem2)
    c1.wait()
    c2.wait()
    pltpu.sync_copy(rows1_v, emb_hbm.at[pl.ds(base, _BPW)])
    pltpu.sync_copy(rows2_v, rest_hbm.at[pl.ds(base, _BPW)])


@functools.cache
def _sc_stage():
  return pl.kernel(
    _sc_body,
    mesh=plsc.VectorSubcoreMesh(core_axis_name="c", subcore_axis_name="s"),
    compiler_params=pltpu.CompilerParams(use_tc_tiling_on_sc=False),
    out_type=[
        jax.ShapeDtypeStruct((_B, _DIM), jnp.float32),
        jax.ShapeDtypeStruct((_B, _DIM), jnp.float32),
    ],
    scratch_types=[
        pltpu.VMEM((_BPW,), jnp.int32),
        pltpu.VMEM((_BPW,), jnp.int32),
        pltpu.VMEM((_BPW, _DIM), jnp.float32),
        pltpu.VMEM((_BPW, _DIM), jnp.float32),
        pltpu.SemaphoreType.DMA,
        pltpu.SemaphoreType.DMA,
    ],
  )


def _mlp_body(emb_ref, rest_ref, scal_ref, w1a_ref, w1s_ref, b1_ref,
              w2_ref, b2_ref, out_ref):
    pre = jnp.dot(emb_ref[...], w1a_ref[...],
                  preferred_element_type=jnp.float32)
    pre = pre + jnp.dot(scal_ref[...], w1s_ref[...],
                        preferred_element_type=jnp.float32)
    pre = pre + rest_ref[...] + b1_ref[...]
    h = jnp.maximum(pre, jnp.float32(0.0))
    out_ref[...] = jnp.dot(h, w2_ref[...],
                           preferred_element_type=jnp.float32) + b2_ref[...]


def _mlp(emb, rest, scal8, w1a, w1s8, b1, w2, b2):
    row_spec = pl.BlockSpec((_BLK, _DIM), lambda i: (i, 0))
    full = lambda shape: pl.BlockSpec(shape, lambda i: (0, 0))
    return pl.pallas_call(
        _mlp_body,
        grid=(_B // _BLK,),
        in_specs=[
            row_spec, row_spec,
            pl.BlockSpec((_BLK, 8), lambda i: (i, 0)),
            full((_DIM, _DIM)), full((8, _DIM)), full((1, _DIM)),
            full((_DIM, _DIM)), full((1, _DIM)),
        ],
        out_specs=row_spec,
        out_shape=jax.ShapeDtypeStruct((_B, _DIM), jnp.float32),
    )(emb, rest, scal8, w1a, w1s8, b1, w2, b2)


def kernel(user_id, age, sin_month, cos_month, view_count, click_count,
           gender, country, user_table, W1, b1, W2, b2):
    inv = jnp.float32(_INV)
    idx1 = user_id.astype(jnp.int32)
    idx2 = (gender * 32 + country + _VOCAB1).astype(jnp.int32)
    gc = (W1[21:24][:, None, :] + W1[24:56][None, :, :]).reshape(96, _DIM)
    table2 = jnp.concatenate([user_table, gc], axis=0)
    scal8 = jnp.stack([age, sin_month, cos_month, view_count, click_count,
                       jnp.zeros_like(age), jnp.zeros_like(age),
                       jnp.zeros_like(age)], axis=1) * inv
    w1s8 = jnp.pad(W1[16:21], ((0, 3), (0, 0)))
    emb, rest = _sc_stage()(table2, idx1, idx2)
    return _mlp(emb, rest, scal8, w1a=W1[:_DIM], w1s8=w1s8,
                b1=b1.reshape(1, _DIM), w2=W2, b2=b2.reshape(1, _DIM))


# speedup vs baseline: 1.6062x; 1.6062x over previous
import functools

import jax
import jax.numpy as jnp
from jax import lax
from jax.experimental import pallas as pl
from jax.experimental.pallas import tpu as pltpu
from jax.experimental.pallas import tpu_sc as plsc

_B = 16384
_DIM = 16
_VOCAB_PAD = 1008
_NC = 2
_NS = 16
_NW = _NC * _NS
_BPW = _B // _NW
_INV = 1.0 / (1.0 + 1e-6)
_BLK = 2048


def _sc_body(t_hbm, uid_hbm, embt_hbm, uid_v, t_v, embt_v):
    wid = lax.axis_index("s") * _NC + lax.axis_index("c")
    base = wid * _BPW

    pltpu.sync_copy(uid_hbm.at[pl.ds(base, _BPW)], uid_v)
    pltpu.sync_copy(t_hbm, t_v)

    def body(blk, carry):
        b16 = blk * 16
        src = uid_v[pl.ds(b16, 16)] * 16
        for f in range(16):
            embt_v[f, pl.ds(b16, 16)] = plsc.load_gather(t_v, [src + f])
        return carry

    lax.fori_loop(0, _BPW // 16, body, 0)

    pltpu.sync_copy(embt_v, embt_hbm.at[:, pl.ds(base, _BPW)])


@functools.cache
def _sc_stage():
  return pl.kernel(
    _sc_body,
    mesh=plsc.VectorSubcoreMesh(core_axis_name="c", subcore_axis_name="s"),
    compiler_params=pltpu.CompilerParams(needs_layout_passes=False),
    out_type=jax.ShapeDtypeStruct((_DIM, _B), jnp.float32),
    scratch_types=[
        pltpu.VMEM((_BPW,), jnp.int32),
        pltpu.VMEM((_VOCAB_PAD * _DIM,), jnp.float32),
        pltpu.VMEM((_DIM, _BPW), jnp.float32),
    ],
  )


def _mlp_body(embt_ref, scal_ref, w1_ref, b1_ref, w2_ref, b2_ref, out_ref):
    cdim = (((0,), (0,)), ((), ()))
    w1 = w1_ref[...]
    sv = scal_ref[...]
    ghot = (lax.broadcasted_iota(jnp.int32, (3, _BLK), 0).astype(jnp.float32)
            == sv[5:6, :]).astype(jnp.float32)
    chot = (lax.broadcasted_iota(jnp.int32, (32, _BLK), 0).astype(jnp.float32)
            == sv[6:7, :]).astype(jnp.float32)
    xr = jnp.concatenate([sv[0:5, :], ghot, chot], axis=0)
    pret = lax.dot_general(w1[0:16, :], embt_ref[...], cdim,
                           preferred_element_type=jnp.float32)
    pret = pret + lax.dot_general(w1[16:56, :], xr, cdim,
                                  preferred_element_type=jnp.float32)
    pret = pret + b1_ref[...]
    ht = jnp.maximum(pret, jnp.float32(0.0))
    outt = lax.dot_general(w2_ref[...], ht, cdim,
                           preferred_element_type=jnp.float32)
    out_ref[...] = (outt + b2_ref[...]).T


def _mlp(embt, scal8, w1, b1, w2, b2):
    full = lambda shape: pl.BlockSpec(shape, lambda i: (0, 0))
    return pl.pallas_call(
        _mlp_body,
        grid=(_B // _BLK,),
        in_specs=[
            pl.BlockSpec((_DIM, _BLK), lambda i: (0, i)),
            pl.BlockSpec((8, _BLK), lambda i: (0, i)),
            full((56, _DIM)), full((_DIM, 1)),
            full((_DIM, _DIM)), full((_DIM, 1)),
        ],
        out_specs=pl.BlockSpec((_BLK, _DIM), lambda i: (i, 0)),
        out_shape=jax.ShapeDtypeStruct((_B, _DIM), jnp.float32),
    )(embt, scal8, w1, b1, w2, b2)


def kernel(user_id, age, sin_month, cos_month, view_count, click_count,
           gender, country, user_table, W1, b1, W2, b2):
    inv = jnp.float32(_INV)
    ut = jnp.pad(user_table, ((0, _VOCAB_PAD - user_table.shape[0]), (0, 0)))
    scal8 = jnp.stack([age * inv, sin_month * inv, cos_month * inv,
                       view_count * inv, click_count * inv,
                       gender.astype(jnp.float32),
                       country.astype(jnp.float32),
                       jnp.zeros_like(age)], axis=0)
    embt = _sc_stage()(ut.reshape(-1), user_id.astype(jnp.int32))
    return _mlp(embt, scal8, W1, b1.reshape(_DIM, 1),
                W2, b2.reshape(_DIM, 1))


# speedup vs baseline: 1.6688x; 1.0389x over previous
import functools

import jax
import jax.numpy as jnp
from jax import lax
from jax.experimental import pallas as pl
from jax.experimental.pallas import tpu as pltpu
from jax.experimental.pallas import tpu_sc as plsc

_B = 16384
_DIM = 16
_VOCAB1 = 1001
_NC = 2
_NS = 16
_NW = _NC * _NS
_BPW = _B // _NW
_INV = 1.0 / (1.0 + 1e-6)
_BLK = 2048
_NB = _B // _BLK


def _sc_body(t_hbm, uid_hbm, embt_hbm, uid_v, t_v, embt_v):
    wid = lax.axis_index("s") * _NC + lax.axis_index("c")
    base = wid * _BPW

    pltpu.sync_copy(uid_hbm.at[pl.ds(base, _BPW)], uid_v)
    pltpu.sync_copy(t_hbm, t_v)

    def body(blk, carry):
        b16 = blk * 16
        src = uid_v[pl.ds(b16, 16)] * 16
        for f in range(16):
            embt_v[f, pl.ds(b16, 16)] = plsc.load_gather(t_v, [src + f])
        return carry

    lax.fori_loop(0, _BPW // 16, body, 0)

    pltpu.sync_copy(embt_v, embt_hbm.at[:, pl.ds(base, _BPW)])


@functools.cache
def _sc_stage():
  return pl.kernel(
    _sc_body,
    mesh=plsc.VectorSubcoreMesh(core_axis_name="c", subcore_axis_name="s"),
    compiler_params=pltpu.CompilerParams(needs_layout_passes=False),
    out_type=jax.ShapeDtypeStruct((_DIM, _B), jnp.float32),
    scratch_types=[
        pltpu.VMEM((_BPW,), jnp.int32),
        pltpu.VMEM((_VOCAB1 * _DIM,), jnp.float32),
        pltpu.VMEM((_DIM, _BPW), jnp.float32),
    ],
  )


def _mlp_body(embt_ref, age_ref, sin_ref, cos_ref, vw_ref, ck_ref,
              g_ref, c_ref, w1_ref, b1_ref, w2_ref, b2_ref, out_ref):
    cdim = (((0,), (0,)), ((), ()))
    w1 = w1_ref[...]
    scal5 = jnp.concatenate(
        [age_ref[0], sin_ref[0], cos_ref[0], vw_ref[0], ck_ref[0]], axis=0)
    ghot = (lax.broadcasted_iota(jnp.int32, (3, _BLK), 0)
            == g_ref[0]).astype(jnp.float32)
    chot = (lax.broadcasted_iota(jnp.int32, (32, _BLK), 0)
            == c_ref[0]).astype(jnp.float32)
    pret = lax.dot_general(w1[0:16, :], embt_ref[...], cdim,
                           preferred_element_type=jnp.float32)
    pret = pret + lax.dot_general(w1[16:21, :] * jnp.float32(_INV), scal5,
                                  cdim, preferred_element_type=jnp.float32)
    pret = pret + lax.dot_general(w1[21:24, :], ghot, cdim,
                                  preferred_element_type=jnp.float32)
    pret = pret + lax.dot_general(w1[24:56, :], chot, cdim,
                                  preferred_element_type=jnp.float32)
    pret = pret + b1_ref[...]
    ht = jnp.maximum(pret, jnp.float32(0.0))
    outt = lax.dot_general(w2_ref[...], ht, cdim,
                           preferred_element_type=jnp.float32)
    out_ref[...] = (outt + b2_ref[...]).T


def _mlp(embt, age, sin_month, cos_month, view_count, click_count,
         gender, country, w1, b1, w2, b2):
    vec_spec = pl.BlockSpec((1, 1, _BLK), lambda i: (i, 0, 0))
    full = lambda shape: pl.BlockSpec(shape, lambda i: (0, 0))
    v3 = lambda x: x.reshape(_NB, 1, _BLK)
    return pl.pallas_call(
        _mlp_body,
        grid=(_NB,),
        in_specs=[
            pl.BlockSpec((_DIM, _BLK), lambda i: (0, i)),
            vec_spec, vec_spec, vec_spec, vec_spec, vec_spec,
            vec_spec, vec_spec,
            full((56, _DIM)), full((_DIM, 1)),
            full((_DIM, _DIM)), full((_DIM, 1)),
        ],
        out_specs=pl.BlockSpec((_BLK, _DIM), lambda i: (i, 0)),
        out_shape=jax.ShapeDtypeStruct((_B, _DIM), jnp.float32),
    )(embt, v3(age), v3(sin_month), v3(cos_month), v3(view_count),
      v3(click_count), v3(gender.astype(jnp.int32)),
      v3(country.astype(jnp.int32)), w1, b1, w2, b2)


def kernel(user_id, age, sin_month, cos_month, view_count, click_count,
           gender, country, user_table, W1, b1, W2, b2):
    embt = _sc_stage()(user_table.reshape(-1), user_id.astype(jnp.int32))
    return _mlp(embt, age, sin_month, cos_month, view_count, click_count,
                gender, country, W1, b1.reshape(_DIM, 1),
                W2, b2.reshape(_DIM, 1))


# speedup vs baseline: 1.7381x; 1.0416x over previous
import functools

import jax
import jax.numpy as jnp
from jax import lax
from jax.experimental import pallas as pl
from jax.experimental.pallas import tpu as pltpu
from jax.experimental.pallas import tpu_sc as plsc

_B = 16384
_DIM = 16
_VOCAB1 = 1001
_NC = 2
_NS = 16
_NW = _NC * _NS
_BPW = _B // _NW
_INV = 1.0 / (1.0 + 1e-6)
_BLK = 4096
_NB = _B // _BLK


def _sc_body(t_hbm, uid_hbm, embt_hbm, uid_v, t_v, embt_v):
    wid = lax.axis_index("s") * _NC + lax.axis_index("c")
    base = wid * _BPW

    pltpu.sync_copy(uid_hbm.at[pl.ds(base, _BPW)], uid_v)
    pltpu.sync_copy(t_hbm, t_v)

    def body(blk, carry):
        b16 = blk * 16
        src = uid_v[pl.ds(b16, 16)] * 16
        for f in range(16):
            embt_v[f, pl.ds(b16, 16)] = plsc.load_gather(t_v, [src + f])
        return carry

    lax.fori_loop(0, _BPW // 16, body, 0)

    pltpu.sync_copy(embt_v, embt_hbm.at[:, pl.ds(base, _BPW)])


@functools.cache
def _sc_stage():
  return pl.kernel(
    _sc_body,
    mesh=plsc.VectorSubcoreMesh(core_axis_name="c", subcore_axis_name="s"),
    compiler_params=pltpu.CompilerParams(needs_layout_passes=False),
    out_type=jax.ShapeDtypeStruct((_DIM, _B), jnp.float32),
    scratch_types=[
        pltpu.VMEM((_BPW,), jnp.int32),
        pltpu.VMEM((_VOCAB1 * _DIM,), jnp.float32),
        pltpu.VMEM((_DIM, _BPW), jnp.float32),
    ],
  )


def _mlp_body(embt_ref, age_ref, sin_ref, cos_ref, vw_ref, ck_ref,
              g_ref, c_ref, w1_ref, b1_ref, w2_ref, b2_ref, out_ref):
    cdim = (((0,), (0,)), ((), ()))
    w1 = w1_ref[...]
    scal5 = jnp.concatenate(
        [age_ref[0], sin_ref[0], cos_ref[0], vw_ref[0], ck_ref[0]], axis=0)
    ghot = (lax.broadcasted_iota(jnp.int32, (3, _BLK), 0)
            == g_ref[0]).astype(jnp.float32)
    chot = (lax.broadcasted_iota(jnp.int32, (32, _BLK), 0)
            == c_ref[0]).astype(jnp.float32)
    pret = lax.dot_general(w1[0:16, :], embt_ref[...], cdim,
                           preferred_element_type=jnp.float32)
    pret = pret + lax.dot_general(w1[16:21, :] * jnp.float32(_INV), scal5,
                                  cdim, preferred_element_type=jnp.float32)
    pret = pret + lax.dot_general(w1[21:24, :], ghot, cdim,
                                  preferred_element_type=jnp.float32)
    pret = pret + lax.dot_general(w1[24:56, :], chot, cdim,
                                  preferred_element_type=jnp.float32)
    pret = pret + b1_ref[...]
    ht = jnp.maximum(pret, jnp.float32(0.0))
    outt = lax.dot_general(w2_ref[...], ht, cdim,
                           preferred_element_type=jnp.float32)
    out_ref[...] = (outt + b2_ref[...]).T


def _mlp(embt, age, sin_month, cos_month, view_count, click_count,
         gender, country, w1, b1, w2, b2):
    vec_spec = pl.BlockSpec((1, 1, _BLK), lambda i: (i, 0, 0))
    full = lambda shape: pl.BlockSpec(shape, lambda i: (0, 0))
    v3 = lambda x: x.reshape(_NB, 1, _BLK)
    return pl.pallas_call(
        _mlp_body,
        grid=(_NB,),
        in_specs=[
            pl.BlockSpec((_DIM, _BLK), lambda i: (0, i)),
            vec_spec, vec_spec, vec_spec, vec_spec, vec_spec,
            vec_spec, vec_spec,
            full((56, _DIM)), full((_DIM, 1)),
            full((_DIM, _DIM)), full((_DIM, 1)),
        ],
        out_specs=pl.BlockSpec((_BLK, _DIM), lambda i: (i, 0)),
        out_shape=jax.ShapeDtypeStruct((_B, _DIM), jnp.float32),
    )(embt, v3(age), v3(sin_month), v3(cos_month), v3(view_count),
      v3(click_count), v3(gender.astype(jnp.int32)),
      v3(country.astype(jnp.int32)), w1, b1, w2, b2)


def kernel(user_id, age, sin_month, cos_month, view_count, click_count,
           gender, country, user_table, W1, b1, W2, b2):
    embt = _sc_stage()(user_table.reshape(-1), user_id.astype(jnp.int32))
    return _mlp(embt, age, sin_month, cos_month, view_count, click_count,
                gender, country, W1, b1.reshape(_DIM, 1),
                W2, b2.reshape(_DIM, 1))


# speedup vs baseline: 1.7897x; 1.0297x over previous
import functools

import jax
import jax.numpy as jnp
from jax import lax
from jax.experimental import pallas as pl
from jax.experimental.pallas import tpu as pltpu
from jax.experimental.pallas import tpu_sc as plsc

_B = 16384
_DIM = 16
_VOCAB1 = 1001
_NC = 2
_NS = 16
_NW = _NC * _NS
_BPW = _B // _NW
_INV = 1.0 / (1.0 + 1e-6)
_BLK = 8192
_NB = _B // _BLK


def _sc_body(t_hbm, uid_hbm, embt_hbm, uid_v, t_v, embt_v):
    wid = lax.axis_index("s") * _NC + lax.axis_index("c")
    base = wid * _BPW

    pltpu.sync_copy(uid_hbm.at[pl.ds(base, _BPW)], uid_v)
    pltpu.sync_copy(t_hbm, t_v)

    def body(blk, carry):
        b16 = blk * 16
        src = uid_v[pl.ds(b16, 16)] * 16
        for f in range(16):
            embt_v[f, pl.ds(b16, 16)] = plsc.load_gather(t_v, [src + f])
        return carry

    lax.fori_loop(0, _BPW // 16, body, 0)

    pltpu.sync_copy(embt_v, embt_hbm.at[:, pl.ds(base, _BPW)])


@functools.cache
def _sc_stage():
  return pl.kernel(
    _sc_body,
    mesh=plsc.VectorSubcoreMesh(core_axis_name="c", subcore_axis_name="s"),
    compiler_params=pltpu.CompilerParams(needs_layout_passes=False),
    out_type=jax.ShapeDtypeStruct((_DIM, _B), jnp.float32),
    scratch_types=[
        pltpu.VMEM((_BPW,), jnp.int32),
        pltpu.VMEM((_VOCAB1 * _DIM,), jnp.float32),
        pltpu.VMEM((_DIM, _BPW), jnp.float32),
    ],
  )


def _mlp_body(embt_ref, age_ref, sin_ref, cos_ref, vw_ref, ck_ref,
              g_ref, c_ref, w1_ref, b1_ref, w2_ref, b2_ref, out_ref):
    cdim = (((0,), (0,)), ((), ()))
    w1 = w1_ref[...]
    scal5 = jnp.concatenate(
        [age_ref[0], sin_ref[0], cos_ref[0], vw_ref[0], ck_ref[0]], axis=0)
    ghot = (lax.broadcasted_iota(jnp.int32, (3, _BLK), 0)
            == g_ref[0]).astype(jnp.float32)
    chot = (lax.broadcasted_iota(jnp.int32, (32, _BLK), 0)
            == c_ref[0]).astype(jnp.float32)
    pret = lax.dot_general(w1[0:16, :], embt_ref[...], cdim,
                           preferred_element_type=jnp.float32)
    pret = pret + lax.dot_general(w1[16:21, :] * jnp.float32(_INV), scal5,
                                  cdim, preferred_element_type=jnp.float32)
    pret = pret + lax.dot_general(w1[21:24, :], ghot, cdim,
                                  preferred_element_type=jnp.float32)
    pret = pret + lax.dot_general(w1[24:56, :], chot, cdim,
                                  preferred_element_type=jnp.float32)
    pret = pret + b1_ref[...]
    ht = jnp.maximum(pret, jnp.float32(0.0))
    outt = lax.dot_general(w2_ref[...], ht, cdim,
                           preferred_element_type=jnp.float32)
    out_ref[...] = (outt + b2_ref[...]).T


def _mlp(embt, age, sin_month, cos_month, view_count, click_count,
         gender, country, w1, b1, w2, b2):
    vec_spec = pl.BlockSpec((1, 1, _BLK), lambda i: (i, 0, 0))
    full = lambda shape: pl.BlockSpec(shape, lambda i: (0, 0))
    v3 = lambda x: x.reshape(_NB, 1, _BLK)
    return pl.pallas_call(
        _mlp_body,
        grid=(_NB,),
        in_specs=[
            pl.BlockSpec((_DIM, _BLK), lambda i: (0, i)),
            vec_spec, vec_spec, vec_spec, vec_spec, vec_spec,
            vec_spec, vec_spec,
            full((56, _DIM)), full((_DIM, 1)),
            full((_DIM, _DIM)), full((_DIM, 1)),
        ],
        out_specs=pl.BlockSpec((_BLK, _DIM), lambda i: (i, 0)),
        out_shape=jax.ShapeDtypeStruct((_B, _DIM), jnp.float32),
    )(embt, v3(age), v3(sin_month), v3(cos_month), v3(view_count),
      v3(click_count), v3(gender.astype(jnp.int32)),
      v3(country.astype(jnp.int32)), w1, b1, w2, b2)


def kernel(user_id, age, sin_month, cos_month, view_count, click_count,
           gender, country, user_table, W1, b1, W2, b2):
    embt = _sc_stage()(user_table.reshape(-1), user_id.astype(jnp.int32))
    return _mlp(embt, age, sin_month, cos_month, view_count, click_count,
                gender, country, W1, b1.reshape(_DIM, 1),
                W2, b2.reshape(_DIM, 1))
